# baseline (speedup 1.0000x reference)
import functools
import jax
import jax.numpy as jnp
from jax import lax
from jax.experimental import pallas as pl
from jax.experimental.pallas import tpu as pltpu
from jax.experimental.pallas import tpu_sc as plsc

N = 10000
M = 200
E = 320000
E_CG = 3200
D = 128
N_RBF = 16
F_VEC = 50
VOCAB = 100
MP = 256
TE = 2560
TN = 1000

_INV_STEP = 15.0 / 5.0

_SC_CORES = 2
_SC_SUBCORES = 16
_SC_WORKERS = _SC_CORES * _SC_SUBCORES
E2 = 327680
GROUPS = E2 // 128
NROWS = N + 16
NPT = NROWS // _SC_SUBCORES


GW = 16
GWO = GW
GCH_G = 2
GCHUNK = GCH_G * 128
EH = E2 // 2
GROUPS_H = GROUPS // 2
GPW = GROUPS_H // _SC_WORKERS


def _geom_body(nxyz16_hbm, src2d_hbm, dst2d_hbm, gs_hbm, gd_hbm,
               sidx, didx, rows_s, rows_d, sem_s, sem_d):
    c = lax.axis_index("c")
    s = lax.axis_index("s")
    wgrp = (c * _SC_SUBCORES + s) * GPW

    def body(i, carry):
        gr = wgrp + i * GCH_G
        pltpu.sync_copy(src2d_hbm.at[pl.ds(gr, GCH_G)], sidx)
        pltpu.sync_copy(dst2d_hbm.at[pl.ds(gr, GCH_G)], didx)
        cps = []
        for j in range(GCH_G):
            sl = pl.ds(j * 128, 128)
            cps.append(pltpu.async_copy(
                nxyz16_hbm.at[sidx.at[j]], rows_s.at[sl], sem_s))
            cps.append(pltpu.async_copy(
                nxyz16_hbm.at[didx.at[j]], rows_d.at[sl], sem_d))
        for cp in cps:
            cp.wait()
        pltpu.sync_copy(rows_s, gs_hbm.at[pl.ds(gr * 128, GCHUNK)])
        pltpu.sync_copy(rows_d, gd_hbm.at[pl.ds(gr * 128, GCHUNK)])
        return carry

    lax.fori_loop(0, GPW // GCH_G, body, 0)


def _geom_call(nxyz16, src2d, dst2d):
    mesh = plsc.VectorSubcoreMesh(core_axis_name="c", subcore_axis_name="s")
    f = functools.partial(
        pl.kernel, _geom_body, mesh=mesh,
        compiler_params=pltpu.CompilerParams(use_tc_tiling_on_sc=False),
        out_type=(jax.ShapeDtypeStruct((EH, GWO), jnp.float32),
                  jax.ShapeDtypeStruct((EH, GWO), jnp.float32)),
        scratch_types=[
            pltpu.VMEM((GCH_G, 128), jnp.int32),
            pltpu.VMEM((GCH_G, 128), jnp.int32),
            pltpu.VMEM((GCHUNK, GW), jnp.float32),
            pltpu.VMEM((GCHUNK, GW), jnp.float32),
            pltpu.SemaphoreType.DMA,
            pltpu.SemaphoreType.DMA,
        ],
    )()
    return f(nxyz16, src2d, dst2d)


DH = D // 2
SCH_G = 2
GPT = GROUPS_H // _SC_SUBCORES
NCH = GPT // SCH_G


def _scatter_body(msg1_hbm, msg2_hbm, d2d1_hbm, d2d2_hbm, zeros_hbm,
                  agg_hbm, agg, buf0, buf1, idx0, idx1, semf0, semf1):
    c = lax.axis_index("c")
    s = lax.axis_index("s")
    rbase = s * NPT
    col = c * DH
    bufs, idxs, semfs = (buf0, buf1), (idx0, idx1), (semf0, semf1)

    pltpu.sync_copy(zeros_hbm, agg.at[pl.ds(rbase, NPT)])
    plsc.subcore_barrier()

    def half(msg_hbm, d2d_hbm):
        def fetch(k, b):
            r = s * GPT + k * SCH_G
            return (pltpu.make_async_copy(
                        d2d_hbm.at[pl.ds(r, SCH_G)], idxs[b], semfs[b]),
                    pltpu.make_async_copy(
                        msg_hbm.at[pl.ds(r * 128, SCH_G * 128),
                                   pl.ds(col, DH)],
                        bufs[b], semfs[b]))

        for b in range(2):
            for cp in fetch(b, b):
                cp.start()

        def body(i, carry):
            for b in range(2):
                k = 2 * i + b
                for cp in fetch(k, b):
                    cp.wait()
                for j in range(SCH_G):
                    pltpu.sync_copy(bufs[b].at[pl.ds(j * 128, 128)],
                                    agg.at[idxs[b].at[j]], add=True)
                kn = jnp.minimum(k + 2, NCH - 1)
                for cp in fetch(kn, b):
                    cp.start()
            return carry

        lax.fori_loop(0, NCH // 2, body, 0)
        for b in range(2):
            for cp in fetch(0, b):
                cp.wait()

    half(msg1_hbm, d2d1_hbm)
    half(msg2_hbm, d2d2_hbm)
    plsc.subcore_barrier()
    pltpu.sync_copy(agg.at[pl.ds(rbase, NPT)],
                    agg_hbm.at[c, pl.ds(rbase, NPT)])


def _scatter_call(msg1, msg2, d2d1, d2d2, zeros_tile):
    mesh = plsc.VectorSubcoreMesh(core_axis_name="c", subcore_axis_name="s")
    f = functools.partial(
        pl.kernel, _scatter_body, mesh=mesh,
        compiler_params=pltpu.CompilerParams(use_tc_tiling_on_sc=False),
        out_type=jax.ShapeDtypeStruct((_SC_CORES, NROWS, DH), jnp.float32),
        scratch_types=[
            pltpu.VMEM_SHARED((NROWS, DH), jnp.float32),
            pltpu.VMEM((SCH_G * 128, DH), jnp.float32),
            pltpu.VMEM((SCH_G * 128, DH), jnp.float32),
            pltpu.VMEM((SCH_G, 128), jnp.int32),
            pltpu.VMEM((SCH_G, 128), jnp.int32),
            pltpu.SemaphoreType.DMA,
            pltpu.SemaphoreType.DMA,
        ],
    )()
    out = f(msg1, msg2, d2d1, d2d2, zeros_tile)
    return out[0, :N], out[1, :N]


def _centers_row(rows):
    k = lax.broadcasted_iota(jnp.int32, (rows, N_RBF), 1)
    return k.astype(jnp.float32) / _INV_STEP


def _msg_body(gs_ref, gd_ref, emb_ref, wf_ref, bf_ref, out_ref):
    gs = gs_ref[...]
    gd = gd_ref[...]
    dvec = gd[:, 1:4] - gs[:, 1:4]
    d = jnp.sqrt(jnp.sum(dvec * dvec, axis=1, keepdims=True))
    a0 = gs[:, 0:1]
    z = jnp.clip(jnp.abs(a0 * 10.0).astype(jnp.int32), 0, VOCAB - 1)
    lane = lax.broadcasted_iota(jnp.int32, (TE, D), 1)
    onehot = (lane == z).astype(jnp.float32)
    base = jnp.dot(onehot, emb_ref[...], preferred_element_type=jnp.float32)
    rbf = jnp.exp(-2.0 * (d - _centers_row(TE)) ** 2)
    filt = jnp.dot(rbf, wf_ref[...], preferred_element_type=jnp.float32)
    filt = filt + bf_ref[...]
    out_ref[...] = base * filt


def _msg_call(gs, gd, emb128, W_f, b_f):
    grid = EH // TE
    return pl.pallas_call(
        _msg_body,
        grid=(grid,),
        in_specs=[
            pl.BlockSpec((TE, GWO), lambda i: (i, 0)),
            pl.BlockSpec((TE, GWO), lambda i: (i, 0)),
            pl.BlockSpec((D, D), lambda i: (0, 0)),
            pl.BlockSpec((N_RBF, D), lambda i: (0, 0)),
            pl.BlockSpec((1, D), lambda i: (0, 0)),
        ],
        out_specs=pl.BlockSpec((TE, D), lambda i: (i, 0)),
        out_shape=jax.ShapeDtypeStruct((EH, D), jnp.float32),
    )(gs, gd, emb128, W_f, b_f)


def _atom_body(p0_ref, p1_ref, a0_ref, map_ref, emb_ref, wu_ref, bu_ref,
               si_ref, acc_ref):
    step = pl.program_id(0)
    agg = jnp.concatenate([p0_ref[...], p1_ref[...]], axis=1)
    a0 = a0_ref[...]
    z = jnp.clip(jnp.abs(a0 * 10.0).astype(jnp.int32), 0, VOCAB - 1)
    lane = lax.broadcasted_iota(jnp.int32, (TN, D), 1)
    onehot = (lane == z).astype(jnp.float32)
    h0 = jnp.dot(onehot, emb_ref[...], preferred_element_type=jnp.float32)
    u = jnp.dot(agg, wu_ref[...], preferred_element_type=jnp.float32)
    h = h0 + jnp.maximum(u + bu_ref[...], 0.0)
    cgm = map_ref[...]
    lane_m = lax.broadcasted_iota(jnp.int32, (TN, MP), 1)
    onehot_cg = (lane_m == cgm).astype(jnp.float32)
    part = lax.dot_general(onehot_cg, h, (((0,), (0,)), ((), ())),
                           preferred_element_type=jnp.float32)

    @pl.when(step == 0)
    def _():
        acc_ref[...] = jnp.zeros_like(acc_ref)

    acc_ref[...] += part
    si_ref[...] = acc_ref[...]


def _atom_call(p0, p1, a0_col, map_col, emb128, W_u, b_u):
    grid = N // TN
    return pl.pallas_call(
        _atom_body,
        grid=(grid,),
        in_specs=[
            pl.BlockSpec((TN, DH), lambda i: (i, 0)),
            pl.BlockSpec((TN, DH), lambda i: (i, 0)),
            pl.BlockSpec((TN, 1), lambda i: (i, 0)),
            pl.BlockSpec((TN, 1), lambda i: (i, 0)),
            pl.BlockSpec((D, D), lambda i: (0, 0)),
            pl.BlockSpec((D, D), lambda i: (0, 0)),
            pl.BlockSpec((1, D), lambda i: (0, 0)),
        ],
        out_specs=pl.BlockSpec((MP, D), lambda i: (0, 0)),
        out_shape=jax.ShapeDtypeStruct((MP, D), jnp.float32),
        scratch_shapes=[pltpu.VMEM((MP, D), jnp.float32)],
    )(p0, p1, a0_col, map_col, emb128, W_u, b_u)


def _cg_body(si_ref, eps_ref, ci_ref, cj_ref, cgp_ref,
             wmu1_ref, bmu1_ref, wmu2_ref, bmu2_ref,
             wsg1_ref, bsg1_ref, wsg2_ref, bsg2_ref,
             wp1a_ref, wp1b_ref, wp1c_ref, bp1_ref, wp2v_ref, bp2v_ref,
             smu_ref, ssig_ref, cgv0_ref, cgv1_ref, cgv2_ref):
    S_I = si_ref[...]
    mu1 = jnp.maximum(jnp.dot(S_I, wmu1_ref[...],
                              preferred_element_type=jnp.float32)
                      + bmu1_ref[...], 0.0)
    S_mu = jnp.dot(mu1, wmu2_ref[...],
                   preferred_element_type=jnp.float32) + bmu2_ref[...]
    sg1 = jnp.maximum(jnp.dot(S_I, wsg1_ref[...],
                              preferred_element_type=jnp.float32)
                      + bsg1_ref[...], 0.0)
    S_logvar = jnp.dot(sg1, wsg2_ref[...],
                       preferred_element_type=jnp.float32) + bsg2_ref[...]
    S_sigma = jnp.exp(S_logvar * 0.5)
    S_lat = eps_ref[...] * S_sigma + S_mu

    ci = ci_ref[...]
    cj = cj_ref[...]
    lane_m = lax.broadcasted_iota(jnp.int32, (E_CG, MP), 1)
    oh_i = (lane_m == ci).astype(jnp.float32)
    oh_j = (lane_m == cj).astype(jnp.float32)
    cgp = cgp_ref[...]
    pi = jnp.dot(oh_i, cgp, preferred_element_type=jnp.float32)
    pj = jnp.dot(oh_j, cgp, preferred_element_type=jnp.float32)
    dvec = pj - pi
    d2 = jnp.sum(dvec * dvec, axis=1, keepdims=True)
    cdist = jnp.sqrt(d2) + 1e-8
    unit = dvec / cdist
    rbf = jnp.exp(-2.0 * (cdist - _centers_row(E_CG)) ** 2)

    Si = jnp.dot(oh_i, S_lat, preferred_element_type=jnp.float32)
    Sj = jnp.dot(oh_j, S_lat, preferred_element_type=jnp.float32)
    pre = (jnp.dot(Si, wp1a_ref[...], preferred_element_type=jnp.float32)
           + jnp.dot(Sj, wp1b_ref[...], preferred_element_type=jnp.float32)
           + jnp.dot(rbf, wp1c_ref[...], preferred_element_type=jnp.float32)
           + bp1_ref[...])
    phi1 = jnp.maximum(pre, 0.0)
    v_w = jnp.dot(phi1, wp2v_ref[...],
                  preferred_element_type=jnp.float32) + bp2v_ref[...]

    lane_d = lax.broadcasted_iota(jnp.int32, (D, 1), 0)
    for c, out in ((0, cgv0_ref), (1, cgv1_ref), (2, cgv2_ref)):
        ec = (lane_d == c).astype(jnp.float32)
        uc = jnp.dot(unit, ec, preferred_element_type=jnp.float32)
        wv = v_w * uc
        out[...] = lax.dot_general(oh_i, wv, (((0,), (0,)), ((), ())),
                                   preferred_element_type=jnp.float32)

    smu_ref[...] = S_mu
    ssig_ref[...] = S_sigma


def _cg_call(S_I, eps_pad, ci_col, cj_col, cgp, weights):
    (W_mu1, b_mu1, W_mu2, b_mu2, W_sg1, b_sg1, W_sg2, b_sg2,
     W_p1a, W_p1b, W_p1c, b_p1, W_p2v, b_p2v) = weights
    out_shape = [jax.ShapeDtypeStruct((MP, D), jnp.float32)] * 5
    return pl.pallas_call(
        _cg_body,
        out_shape=out_shape,
    )(S_I, eps_pad, ci_col, cj_col, cgp,
      W_mu1, b_mu1, W_mu2, b_mu2, W_sg1, b_sg1, W_sg2, b_sg2,
      W_p1a, W_p1b, W_p1c, b_p1, W_p2v, b_p2v)


def _recon_body(cgv_ref, map_ref, cgp_ref, out_ref):
    cgm = map_ref[...]
    lane_m = lax.broadcasted_iota(jnp.int32, (TN, MP), 1)
    onehot = (lane_m == cgm).astype(jnp.float32)
    anchor = jnp.dot(onehot, cgp_ref[...],
                     preferred_element_type=jnp.float32)
    out_ref[...] = cgv_ref[...] + anchor


def _recon_call(cgv_flat_pad, map_col, cgp):
    grid = N // TN
    return pl.pallas_call(
        _recon_body,
        grid=(grid,),
        in_specs=[
            pl.BlockSpec((TN, D), lambda i: (i, 0)),
            pl.BlockSpec((TN, 1), lambda i: (i, 0)),
            pl.BlockSpec((MP, D), lambda i: (0, 0)),
        ],
        out_specs=pl.BlockSpec((TN, D), lambda i: (i, 0)),
        out_shape=jax.ShapeDtypeStruct((N, D), jnp.float32),
    )(cgv_flat_pad, map_col, cgp)


def kernel(nxyz, CG_nxyz, CG_mapping, nbr_list, CG_nbr_list, num_CGs, eps,
           emb, W_f, b_f, W_u, b_u, W_p1, b_p1, W_p2, b_p2,
           W_mu1, b_mu1, W_mu2, b_mu2, W_sg1, b_sg1, W_sg2, b_sg2):
    xyz = nxyz[:, 1:]
    a0_col = nxyz[:, 0:1]
    src = jnp.concatenate(
        [nbr_list[:, 0].astype(jnp.int32), jnp.zeros((E2 - E,), jnp.int32)])
    dst = jnp.concatenate(
        [nbr_list[:, 1].astype(jnp.int32), jnp.full((E2 - E,), N, jnp.int32)])
    src2d = src.reshape(GROUPS, 128)
    dst2d = dst.reshape(GROUPS, 128)
    s2d1, s2d2 = src2d[:GROUPS_H], src2d[GROUPS_H:]
    d2d1, d2d2 = dst2d[:GROUPS_H], dst2d[GROUPS_H:]

    nxyz16 = jnp.zeros((N, GW), jnp.float32).at[:, :4].set(nxyz)
    emb128 = jnp.zeros((D, D), jnp.float32).at[:VOCAB].set(emb)
    gs1, gd1 = _geom_call(nxyz16, s2d1, d2d1)
    gs2, gd2 = _geom_call(nxyz16, s2d2, d2d2)
    msg1 = _msg_call(gs1, gd1, emb128, W_f, b_f[None, :])
    msg2 = _msg_call(gs2, gd2, emb128, W_f, b_f[None, :])

    zeros_tile = jnp.zeros((NPT, DH), jnp.float32)
    agg0, agg1 = _scatter_call(msg1, msg2, d2d1, d2d2, zeros_tile)

    map_col = CG_mapping[:, None].astype(jnp.int32)
    S_I = _atom_call(agg0, agg1, a0_col, map_col, emb128, W_u, b_u[None, :])

    eps_pad = jnp.zeros((MP, D), jnp.float32).at[:M].set(eps)
    cgp = jnp.zeros((MP, D), jnp.float32).at[:M, :3].set(CG_nxyz[:, 1:])
    ci_col = CG_nbr_list[:, 0:1].astype(jnp.int32)
    cj_col = CG_nbr_list[:, 1:2].astype(jnp.int32)
    weights = (W_mu1, b_mu1[None, :], W_mu2, b_mu2[None, :],
               W_sg1, b_sg1[None, :], W_sg2, b_sg2[None, :],
               W_p1[:D], W_p1[D:2 * D], W_p1[2 * D:], b_p1[None, :],
               jnp.zeros((D, D), jnp.float32).at[:, :F_VEC].set(W_p2[:, D:]),
               jnp.zeros((1, D), jnp.float32).at[0, :F_VEC].set(b_p2[D:]))
    S_mu_p, S_sig_p, cgv0, cgv1, cgv2 = _cg_call(
        S_I, eps_pad, ci_col, cj_col, cgp, weights)

    cgv_flat = jnp.stack(
        [cgv0[:M, :F_VEC].reshape(-1),
         cgv1[:M, :F_VEC].reshape(-1),
         cgv2[:M, :F_VEC].reshape(-1)], axis=-1)
    cgv_flat_pad = jnp.zeros((N, D), jnp.float32).at[:, :3].set(cgv_flat)
    recon_pad = _recon_call(cgv_flat_pad, map_col, cgp)

    return (S_mu_p[:M], S_sig_p[:M], xyz, recon_pad[:, :3])


# speedup vs baseline: 1.0304x; 1.0304x over previous
import functools
import jax
import jax.numpy as jnp
from jax import lax
from jax.experimental import pallas as pl
from jax.experimental.pallas import tpu as pltpu
from jax.experimental.pallas import tpu_sc as plsc

N = 10000
M = 200
E = 320000
E_CG = 3200
D = 128
N_RBF = 16
F_VEC = 50
VOCAB = 100
MP = 256
TE = 2560
TN = 1000

_INV_STEP = 15.0 / 5.0

_SC_CORES = 2
_SC_SUBCORES = 16
_SC_WORKERS = _SC_CORES * _SC_SUBCORES
E2 = 327680
GROUPS = E2 // 128
NROWS = N + 16
NPT = NROWS // _SC_SUBCORES


GW = 16
GWO = GW
GCH_G = 4
GCHUNK = GCH_G * 128
EH = E2
GROUPS_H = GROUPS
GPW = GROUPS_H // _SC_WORKERS


def _geom_body(nxyz16_hbm, src2d_hbm, dst2d_hbm, gs_hbm, gd_hbm,
               sidx, didx, rows_s, rows_d, sem_s, sem_d):
    c = lax.axis_index("c")
    s = lax.axis_index("s")
    wgrp = (c * _SC_SUBCORES + s) * GPW

    def body(i, carry):
        gr = wgrp + i * GCH_G
        pltpu.sync_copy(src2d_hbm.at[pl.ds(gr, GCH_G)], sidx)
        pltpu.sync_copy(dst2d_hbm.at[pl.ds(gr, GCH_G)], didx)
        cps = []
        for j in range(GCH_G):
            sl = pl.ds(j * 128, 128)
            cps.append(pltpu.async_copy(
                nxyz16_hbm.at[sidx.at[j]], rows_s.at[sl], sem_s))
            cps.append(pltpu.async_copy(
                nxyz16_hbm.at[didx.at[j]], rows_d.at[sl], sem_d))
        for cp in cps:
            cp.wait()
        pltpu.sync_copy(rows_s, gs_hbm.at[pl.ds(gr * 128, GCHUNK)])
        pltpu.sync_copy(rows_d, gd_hbm.at[pl.ds(gr * 128, GCHUNK)])
        return carry

    lax.fori_loop(0, GPW // GCH_G, body, 0)


def _geom_call(nxyz16, src2d, dst2d):
    mesh = plsc.VectorSubcoreMesh(core_axis_name="c", subcore_axis_name="s")
    f = functools.partial(
        pl.kernel, _geom_body, mesh=mesh,
        compiler_params=pltpu.CompilerParams(use_tc_tiling_on_sc=False),
        out_type=(jax.ShapeDtypeStruct((EH, GWO), jnp.float32),
                  jax.ShapeDtypeStruct((EH, GWO), jnp.float32)),
        scratch_types=[
            pltpu.VMEM((GCH_G, 128), jnp.int32),
            pltpu.VMEM((GCH_G, 128), jnp.int32),
            pltpu.VMEM((GCHUNK, GW), jnp.float32),
            pltpu.VMEM((GCHUNK, GW), jnp.float32),
            pltpu.SemaphoreType.DMA,
            pltpu.SemaphoreType.DMA,
        ],
    )()
    return f(nxyz16, src2d, dst2d)


DH = D // 2
SCH_G = 2
GPT = GROUPS_H // _SC_SUBCORES
NCH = GPT // SCH_G


def _scatter_body(msg_hbm, d2d_hbm, zeros_hbm,
                  agg_hbm, agg, buf0, buf1, idx0, idx1, semf0, semf1):
    c = lax.axis_index("c")
    s = lax.axis_index("s")
    rbase = s * NPT
    col = c * DH
    bufs, idxs, semfs = (buf0, buf1), (idx0, idx1), (semf0, semf1)

    pltpu.sync_copy(zeros_hbm, agg.at[pl.ds(rbase, NPT)])
    plsc.subcore_barrier()

    def fetch(k, b):
        r = s * GPT + k * SCH_G
        return (pltpu.make_async_copy(
                    d2d_hbm.at[pl.ds(r, SCH_G)], idxs[b], semfs[b]),
                pltpu.make_async_copy(
                    msg_hbm.at[pl.ds(r * 128, SCH_G * 128),
                               pl.ds(col, DH)],
                    bufs[b], semfs[b]))

    for b in range(2):
        for cp in fetch(b, b):
            cp.start()

    def body(i, carry):
        for b in range(2):
            k = 2 * i + b
            for cp in fetch(k, b):
                cp.wait()
            for j in range(SCH_G):
                pltpu.sync_copy(bufs[b].at[pl.ds(j * 128, 128)],
                                agg.at[idxs[b].at[j]], add=True)
            kn = jnp.minimum(k + 2, NCH - 1)
            for cp in fetch(kn, b):
                cp.start()
        return carry

    lax.fori_loop(0, NCH // 2, body, 0)
    for b in range(2):
        for cp in fetch(0, b):
            cp.wait()
    plsc.subcore_barrier()
    pltpu.sync_copy(agg.at[pl.ds(rbase, NPT)],
                    agg_hbm.at[c, pl.ds(rbase, NPT)])


def _scatter_call(msg, d2d, zeros_tile):
    mesh = plsc.VectorSubcoreMesh(core_axis_name="c", subcore_axis_name="s")
    f = functools.partial(
        pl.kernel, _scatter_body, mesh=mesh,
        compiler_params=pltpu.CompilerParams(use_tc_tiling_on_sc=False),
        out_type=jax.ShapeDtypeStruct((_SC_CORES, NROWS, DH), jnp.float32),
        scratch_types=[
            pltpu.VMEM_SHARED((NROWS, DH), jnp.float32),
            pltpu.VMEM((SCH_G * 128, DH), jnp.float32),
            pltpu.VMEM((SCH_G * 128, DH), jnp.float32),
            pltpu.VMEM((SCH_G, 128), jnp.int32),
            pltpu.VMEM((SCH_G, 128), jnp.int32),
            pltpu.SemaphoreType.DMA,
            pltpu.SemaphoreType.DMA,
        ],
    )()
    out = f(msg, d2d, zeros_tile)
    return out[0, :N], out[1, :N]


def _centers_row(rows):
    k = lax.broadcasted_iota(jnp.int32, (rows, N_RBF), 1)
    return k.astype(jnp.float32) / _INV_STEP


def _msg_body(gs_ref, gd_ref, emb_ref, wf_ref, bf_ref, out_ref):
    gs = gs_ref[...]
    gd = gd_ref[...]
    dvec = gd[:, 1:4] - gs[:, 1:4]
    d = jnp.sqrt(jnp.sum(dvec * dvec, axis=1, keepdims=True))
    a0 = gs[:, 0:1]
    z = jnp.clip(jnp.abs(a0 * 10.0).astype(jnp.int32), 0, VOCAB - 1)
    lane = lax.broadcasted_iota(jnp.int32, (TE, D), 1)
    onehot = (lane == z).astype(jnp.float32)
    base = jnp.dot(onehot, emb_ref[...], preferred_element_type=jnp.float32)
    rbf = jnp.exp(-2.0 * (d - _centers_row(TE)) ** 2)
    filt = jnp.dot(rbf, wf_ref[...], preferred_element_type=jnp.float32)
    filt = filt + bf_ref[...]
    out_ref[...] = base * filt


def _msg_call(gs, gd, emb128, W_f, b_f):
    grid = EH // TE
    return pl.pallas_call(
        _msg_body,
        grid=(grid,),
        in_specs=[
            pl.BlockSpec((TE, GWO), lambda i: (i, 0)),
            pl.BlockSpec((TE, GWO), lambda i: (i, 0)),
            pl.BlockSpec((D, D), lambda i: (0, 0)),
            pl.BlockSpec((N_RBF, D), lambda i: (0, 0)),
            pl.BlockSpec((1, D), lambda i: (0, 0)),
        ],
        out_specs=pl.BlockSpec((TE, D), lambda i: (i, 0)),
        out_shape=jax.ShapeDtypeStruct((EH, D), jnp.float32),
    )(gs, gd, emb128, W_f, b_f)


def _atom_body(p0_ref, p1_ref, a0_ref, map_ref, emb_ref, wu_ref, bu_ref,
               si_ref, acc_ref):
    step = pl.program_id(0)
    agg = jnp.concatenate([p0_ref[...], p1_ref[...]], axis=1)
    a0 = a0_ref[...]
    z = jnp.clip(jnp.abs(a0 * 10.0).astype(jnp.int32), 0, VOCAB - 1)
    lane = lax.broadcasted_iota(jnp.int32, (TN, D), 1)
    onehot = (lane == z).astype(jnp.float32)
    h0 = jnp.dot(onehot, emb_ref[...], preferred_element_type=jnp.float32)
    u = jnp.dot(agg, wu_ref[...], preferred_element_type=jnp.float32)
    h = h0 + jnp.maximum(u + bu_ref[...], 0.0)
    cgm = map_ref[...]
    lane_m = lax.broadcasted_iota(jnp.int32, (TN, MP), 1)
    onehot_cg = (lane_m == cgm).astype(jnp.float32)
    part = lax.dot_general(onehot_cg, h, (((0,), (0,)), ((), ())),
                           preferred_element_type=jnp.float32)

    @pl.when(step == 0)
    def _():
        acc_ref[...] = jnp.zeros_like(acc_ref)

    acc_ref[...] += part
    si_ref[...] = acc_ref[...]


def _atom_call(p0, p1, a0_col, map_col, emb128, W_u, b_u):
    grid = N // TN
    return pl.pallas_call(
        _atom_body,
        grid=(grid,),
        in_specs=[
            pl.BlockSpec((TN, DH), lambda i: (i, 0)),
            pl.BlockSpec((TN, DH), lambda i: (i, 0)),
            pl.BlockSpec((TN, 1), lambda i: (i, 0)),
            pl.BlockSpec((TN, 1), lambda i: (i, 0)),
            pl.BlockSpec((D, D), lambda i: (0, 0)),
            pl.BlockSpec((D, D), lambda i: (0, 0)),
            pl.BlockSpec((1, D), lambda i: (0, 0)),
        ],
        out_specs=pl.BlockSpec((MP, D), lambda i: (0, 0)),
        out_shape=jax.ShapeDtypeStruct((MP, D), jnp.float32),
        scratch_shapes=[pltpu.VMEM((MP, D), jnp.float32)],
    )(p0, p1, a0_col, map_col, emb128, W_u, b_u)


def _cg_body(si_ref, eps_ref, ci_ref, cj_ref, cgp_ref,
             wmu1_ref, bmu1_ref, wmu2_ref, bmu2_ref,
             wsg1_ref, bsg1_ref, wsg2_ref, bsg2_ref,
             wp1a_ref, wp1b_ref, wp1c_ref, bp1_ref, wp2v_ref, bp2v_ref,
             smu_ref, ssig_ref, cgv0_ref, cgv1_ref, cgv2_ref):
    S_I = si_ref[...]
    mu1 = jnp.maximum(jnp.dot(S_I, wmu1_ref[...],
                              preferred_element_type=jnp.float32)
                      + bmu1_ref[...], 0.0)
    S_mu = jnp.dot(mu1, wmu2_ref[...],
                   preferred_element_type=jnp.float32) + bmu2_ref[...]
    sg1 = jnp.maximum(jnp.dot(S_I, wsg1_ref[...],
                              preferred_element_type=jnp.float32)
                      + bsg1_ref[...], 0.0)
    S_logvar = jnp.dot(sg1, wsg2_ref[...],
                       preferred_element_type=jnp.float32) + bsg2_ref[...]
    S_sigma = jnp.exp(S_logvar * 0.5)
    S_lat = eps_ref[...] * S_sigma + S_mu

    ci = ci_ref[...]
    cj = cj_ref[...]
    lane_m = lax.broadcasted_iota(jnp.int32, (E_CG, MP), 1)
    oh_i = (lane_m == ci).astype(jnp.float32)
    oh_j = (lane_m == cj).astype(jnp.float32)
    cgp = cgp_ref[...]
    pi = jnp.dot(oh_i, cgp, preferred_element_type=jnp.float32)
    pj = jnp.dot(oh_j, cgp, preferred_element_type=jnp.float32)
    dvec = pj - pi
    d2 = jnp.sum(dvec * dvec, axis=1, keepdims=True)
    cdist = jnp.sqrt(d2) + 1e-8
    unit = dvec / cdist
    rbf = jnp.exp(-2.0 * (cdist - _centers_row(E_CG)) ** 2)

    Si = jnp.dot(oh_i, S_lat, preferred_element_type=jnp.float32)
    Sj = jnp.dot(oh_j, S_lat, preferred_element_type=jnp.float32)
    pre = (jnp.dot(Si, wp1a_ref[...], preferred_element_type=jnp.float32)
           + jnp.dot(Sj, wp1b_ref[...], preferred_element_type=jnp.float32)
           + jnp.dot(rbf, wp1c_ref[...], preferred_element_type=jnp.float32)
           + bp1_ref[...])
    phi1 = jnp.maximum(pre, 0.0)
    v_w = jnp.dot(phi1, wp2v_ref[...],
                  preferred_element_type=jnp.float32) + bp2v_ref[...]

    lane_d = lax.broadcasted_iota(jnp.int32, (D, 1), 0)
    for c, out in ((0, cgv0_ref), (1, cgv1_ref), (2, cgv2_ref)):
        ec = (lane_d == c).astype(jnp.float32)
        uc = jnp.dot(unit, ec, preferred_element_type=jnp.float32)
        wv = v_w * uc
        out[...] = lax.dot_general(oh_i, wv, (((0,), (0,)), ((), ())),
                                   preferred_element_type=jnp.float32)

    smu_ref[...] = S_mu
    ssig_ref[...] = S_sigma


def _cg_call(S_I, eps_pad, ci_col, cj_col, cgp, weights):
    (W_mu1, b_mu1, W_mu2, b_mu2, W_sg1, b_sg1, W_sg2, b_sg2,
     W_p1a, W_p1b, W_p1c, b_p1, W_p2v, b_p2v) = weights
    out_shape = [jax.ShapeDtypeStruct((MP, D), jnp.float32)] * 5
    return pl.pallas_call(
        _cg_body,
        out_shape=out_shape,
    )(S_I, eps_pad, ci_col, cj_col, cgp,
      W_mu1, b_mu1, W_mu2, b_mu2, W_sg1, b_sg1, W_sg2, b_sg2,
      W_p1a, W_p1b, W_p1c, b_p1, W_p2v, b_p2v)


def _recon_body(cgv_ref, map_ref, cgp_ref, out_ref):
    cgm = map_ref[...]
    lane_m = lax.broadcasted_iota(jnp.int32, (TN, MP), 1)
    onehot = (lane_m == cgm).astype(jnp.float32)
    anchor = jnp.dot(onehot, cgp_ref[...],
                     preferred_element_type=jnp.float32)
    out_ref[...] = cgv_ref[...] + anchor


def _recon_call(cgv_flat_pad, map_col, cgp):
    grid = N // TN
    return pl.pallas_call(
        _recon_body,
        grid=(grid,),
        in_specs=[
            pl.BlockSpec((TN, D), lambda i: (i, 0)),
            pl.BlockSpec((TN, 1), lambda i: (i, 0)),
            pl.BlockSpec((MP, D), lambda i: (0, 0)),
        ],
        out_specs=pl.BlockSpec((TN, D), lambda i: (i, 0)),
        out_shape=jax.ShapeDtypeStruct((N, D), jnp.float32),
    )(cgv_flat_pad, map_col, cgp)


def kernel(nxyz, CG_nxyz, CG_mapping, nbr_list, CG_nbr_list, num_CGs, eps,
           emb, W_f, b_f, W_u, b_u, W_p1, b_p1, W_p2, b_p2,
           W_mu1, b_mu1, W_mu2, b_mu2, W_sg1, b_sg1, W_sg2, b_sg2):
    xyz = nxyz[:, 1:]
    a0_col = nxyz[:, 0:1]
    src = jnp.concatenate(
        [nbr_list[:, 0].astype(jnp.int32), jnp.zeros((E2 - E,), jnp.int32)])
    dst = jnp.concatenate(
        [nbr_list[:, 1].astype(jnp.int32), jnp.full((E2 - E,), N, jnp.int32)])
    src2d = src.reshape(GROUPS, 128)
    dst2d = dst.reshape(GROUPS, 128)

    nxyz16 = jnp.zeros((N, GW), jnp.float32).at[:, :4].set(nxyz)
    emb128 = jnp.zeros((D, D), jnp.float32).at[:VOCAB].set(emb)
    gs, gd = _geom_call(nxyz16, src2d, dst2d)

    msg = _msg_call(gs, gd, emb128, W_f, b_f[None, :])

    zeros_tile = jnp.zeros((NPT, DH), jnp.float32)
    agg0, agg1 = _scatter_call(msg, dst2d, zeros_tile)

    map_col = CG_mapping[:, None].astype(jnp.int32)
    S_I = _atom_call(agg0, agg1, a0_col, map_col, emb128, W_u, b_u[None, :])

    eps_pad = jnp.zeros((MP, D), jnp.float32).at[:M].set(eps)
    cgp = jnp.zeros((MP, D), jnp.float32).at[:M, :3].set(CG_nxyz[:, 1:])
    ci_col = CG_nbr_list[:, 0:1].astype(jnp.int32)
    cj_col = CG_nbr_list[:, 1:2].astype(jnp.int32)
    weights = (W_mu1, b_mu1[None, :], W_mu2, b_mu2[None, :],
               W_sg1, b_sg1[None, :], W_sg2, b_sg2[None, :],
               W_p1[:D], W_p1[D:2 * D], W_p1[2 * D:], b_p1[None, :],
               jnp.zeros((D, D), jnp.float32).at[:, :F_VEC].set(W_p2[:, D:]),
               jnp.zeros((1, D), jnp.float32).at[0, :F_VEC].set(b_p2[D:]))
    S_mu_p, S_sig_p, cgv0, cgv1, cgv2 = _cg_call(
        S_I, eps_pad, ci_col, cj_col, cgp, weights)

    cgv_flat = jnp.stack(
        [cgv0[:M, :F_VEC].reshape(-1),
         cgv1[:M, :F_VEC].reshape(-1),
         cgv2[:M, :F_VEC].reshape(-1)], axis=-1)
    cgv_flat_pad = jnp.zeros((N, D), jnp.float32).at[:, :3].set(cgv_flat)
    recon_pad = _recon_call(cgv_flat_pad, map_col, cgp)

    return (S_mu_p[:M], S_sig_p[:M], xyz, recon_pad[:, :3])


# speedup vs baseline: 1.0871x; 1.0549x over previous
import functools
import jax
import jax.numpy as jnp
from jax import lax
from jax.experimental import pallas as pl
from jax.experimental.pallas import tpu as pltpu
from jax.experimental.pallas import tpu_sc as plsc

N = 10000
M = 200
E = 320000
E_CG = 3200
D = 128
N_RBF = 16
F_VEC = 50
VOCAB = 100
MP = 256
TE = 5120
TN = 1000

_INV_STEP = 15.0 / 5.0

_SC_CORES = 2
_SC_SUBCORES = 16
_SC_WORKERS = _SC_CORES * _SC_SUBCORES
E2 = 327680
GROUPS = E2 // 128
NROWS = N + 16
NPT = NROWS // _SC_SUBCORES


GW = 16
GWO = GW
GCH_G = 4
GCHUNK = GCH_G * 128
EH = E2
GROUPS_H = GROUPS
GPW = GROUPS_H // _SC_WORKERS


def _geom_body(nxyz16_hbm, src2d_hbm, dst2d_hbm, gs_hbm, gd_hbm,
               sidx, didx, rows_s, rows_d, sem_s, sem_d):
    c = lax.axis_index("c")
    s = lax.axis_index("s")
    wgrp = (c * _SC_SUBCORES + s) * GPW

    def body(i, carry):
        gr = wgrp + i * GCH_G
        pltpu.sync_copy(src2d_hbm.at[pl.ds(gr, GCH_G)], sidx)
        pltpu.sync_copy(dst2d_hbm.at[pl.ds(gr, GCH_G)], didx)
        cps = []
        for j in range(GCH_G):
            sl = pl.ds(j * 128, 128)
            cps.append(pltpu.async_copy(
                nxyz16_hbm.at[sidx.at[j]], rows_s.at[sl], sem_s))
            cps.append(pltpu.async_copy(
                nxyz16_hbm.at[didx.at[j]], rows_d.at[sl], sem_d))
        for cp in cps:
            cp.wait()
        pltpu.sync_copy(rows_s, gs_hbm.at[pl.ds(gr * 128, GCHUNK)])
        pltpu.sync_copy(rows_d, gd_hbm.at[pl.ds(gr * 128, GCHUNK)])
        return carry

    lax.fori_loop(0, GPW // GCH_G, body, 0)


def _geom_call(nxyz16, src2d, dst2d):
    mesh = plsc.VectorSubcoreMesh(core_axis_name="c", subcore_axis_name="s")
    f = functools.partial(
        pl.kernel, _geom_body, mesh=mesh,
        compiler_params=pltpu.CompilerParams(use_tc_tiling_on_sc=False),
        out_type=(jax.ShapeDtypeStruct((EH, GWO), jnp.float32),
                  jax.ShapeDtypeStruct((EH, GWO), jnp.float32)),
        scratch_types=[
            pltpu.VMEM((GCH_G, 128), jnp.int32),
            pltpu.VMEM((GCH_G, 128), jnp.int32),
            pltpu.VMEM((GCHUNK, GW), jnp.float32),
            pltpu.VMEM((GCHUNK, GW), jnp.float32),
            pltpu.SemaphoreType.DMA,
            pltpu.SemaphoreType.DMA,
        ],
    )()
    return f(nxyz16, src2d, dst2d)


DH = D // 2
SCH_G = 2
GPT = GROUPS_H // _SC_SUBCORES
NCH = GPT // SCH_G


def _scatter_body(msg_hbm, d2d_hbm, zeros_hbm,
                  agg_hbm, agg, buf0, buf1, idx0, idx1, semf0, semf1):
    c = lax.axis_index("c")
    s = lax.axis_index("s")
    rbase = s * NPT
    col = c * DH
    bufs, idxs, semfs = (buf0, buf1), (idx0, idx1), (semf0, semf1)

    pltpu.sync_copy(zeros_hbm, agg.at[pl.ds(rbase, NPT)])
    plsc.subcore_barrier()

    def fetch(k, b):
        r = s * GPT + k * SCH_G
        return (pltpu.make_async_copy(
                    d2d_hbm.at[pl.ds(r, SCH_G)], idxs[b], semfs[b]),
                pltpu.make_async_copy(
                    msg_hbm.at[pl.ds(r * 128, SCH_G * 128),
                               pl.ds(col, DH)],
                    bufs[b], semfs[b]))

    for b in range(2):
        for cp in fetch(b, b):
            cp.start()

    def body(i, carry):
        for b in range(2):
            k = 2 * i + b
            for cp in fetch(k, b):
                cp.wait()
            for j in range(SCH_G):
                pltpu.sync_copy(bufs[b].at[pl.ds(j * 128, 128)],
                                agg.at[idxs[b].at[j]], add=True)
            kn = jnp.minimum(k + 2, NCH - 1)
            for cp in fetch(kn, b):
                cp.start()
        return carry

    lax.fori_loop(0, NCH // 2, body, 0)
    for b in range(2):
        for cp in fetch(0, b):
            cp.wait()
    plsc.subcore_barrier()
    pltpu.sync_copy(agg.at[pl.ds(rbase, NPT)],
                    agg_hbm.at[c, pl.ds(rbase, NPT)])


def _scatter_call(msg, d2d, zeros_tile):
    mesh = plsc.VectorSubcoreMesh(core_axis_name="c", subcore_axis_name="s")
    f = functools.partial(
        pl.kernel, _scatter_body, mesh=mesh,
        compiler_params=pltpu.CompilerParams(use_tc_tiling_on_sc=False),
        out_type=jax.ShapeDtypeStruct((_SC_CORES, NROWS, DH), jnp.float32),
        scratch_types=[
            pltpu.VMEM_SHARED((NROWS, DH), jnp.float32),
            pltpu.VMEM((SCH_G * 128, DH), jnp.float32),
            pltpu.VMEM((SCH_G * 128, DH), jnp.float32),
            pltpu.VMEM((SCH_G, 128), jnp.int32),
            pltpu.VMEM((SCH_G, 128), jnp.int32),
            pltpu.SemaphoreType.DMA,
            pltpu.SemaphoreType.DMA,
        ],
    )()
    out = f(msg, d2d, zeros_tile)
    return out[0, :N], out[1, :N]


def _centers_row(rows):
    k = lax.broadcasted_iota(jnp.int32, (rows, N_RBF), 1)
    return k.astype(jnp.float32) / _INV_STEP


def _msg_body(gs_ref, gd_ref, emb_ref, wf_ref, bf_ref, out_ref):
    gs = gs_ref[...]
    gd = gd_ref[...]
    dvec = gd[:, 1:4] - gs[:, 1:4]
    d = jnp.sqrt(jnp.sum(dvec * dvec, axis=1, keepdims=True))
    a0 = gs[:, 0:1]
    z = jnp.clip(jnp.abs(a0 * 10.0).astype(jnp.int32), 0, VOCAB - 1)
    lane = lax.broadcasted_iota(jnp.int32, (TE, D), 1)
    onehot = (lane == z).astype(jnp.float32)
    base = jnp.dot(onehot, emb_ref[...], preferred_element_type=jnp.float32)
    rbf = jnp.exp(-2.0 * (d - _centers_row(TE)) ** 2)
    filt = jnp.dot(rbf, wf_ref[...], preferred_element_type=jnp.float32)
    filt = filt + bf_ref[...]
    out_ref[...] = base * filt


def _msg_call(gs, gd, emb128, W_f, b_f):
    grid = EH // TE
    return pl.pallas_call(
        _msg_body,
        grid=(grid,),
        in_specs=[
            pl.BlockSpec((TE, GWO), lambda i: (i, 0)),
            pl.BlockSpec((TE, GWO), lambda i: (i, 0)),
            pl.BlockSpec((D, D), lambda i: (0, 0)),
            pl.BlockSpec((N_RBF, D), lambda i: (0, 0)),
            pl.BlockSpec((1, D), lambda i: (0, 0)),
        ],
        out_specs=pl.BlockSpec((TE, D), lambda i: (i, 0)),
        out_shape=jax.ShapeDtypeStruct((EH, D), jnp.float32),
    )(gs, gd, emb128, W_f, b_f)


def _atom_body(p0_ref, p1_ref, a0_ref, map_ref, emb_ref, wu_ref, bu_ref,
               si_ref, acc_ref):
    step = pl.program_id(0)
    agg = jnp.concatenate([p0_ref[...], p1_ref[...]], axis=1)
    a0 = a0_ref[...]
    z = jnp.clip(jnp.abs(a0 * 10.0).astype(jnp.int32), 0, VOCAB - 1)
    lane = lax.broadcasted_iota(jnp.int32, (TN, D), 1)
    onehot = (lane == z).astype(jnp.float32)
    h0 = jnp.dot(onehot, emb_ref[...], preferred_element_type=jnp.float32)
    u = jnp.dot(agg, wu_ref[...], preferred_element_type=jnp.float32)
    h = h0 + jnp.maximum(u + bu_ref[...], 0.0)
    cgm = map_ref[...]
    lane_m = lax.broadcasted_iota(jnp.int32, (TN, MP), 1)
    onehot_cg = (lane_m == cgm).astype(jnp.float32)
    part = lax.dot_general(onehot_cg, h, (((0,), (0,)), ((), ())),
                           preferred_element_type=jnp.float32)

    @pl.when(step == 0)
    def _():
        acc_ref[...] = jnp.zeros_like(acc_ref)

    acc_ref[...] += part
    si_ref[...] = acc_ref[...]


def _atom_call(p0, p1, a0_col, map_col, emb128, W_u, b_u):
    grid = N // TN
    return pl.pallas_call(
        _atom_body,
        grid=(grid,),
        in_specs=[
            pl.BlockSpec((TN, DH), lambda i: (i, 0)),
            pl.BlockSpec((TN, DH), lambda i: (i, 0)),
            pl.BlockSpec((TN, 1), lambda i: (i, 0)),
            pl.BlockSpec((TN, 1), lambda i: (i, 0)),
            pl.BlockSpec((D, D), lambda i: (0, 0)),
            pl.BlockSpec((D, D), lambda i: (0, 0)),
            pl.BlockSpec((1, D), lambda i: (0, 0)),
        ],
        out_specs=pl.BlockSpec((MP, D), lambda i: (0, 0)),
        out_shape=jax.ShapeDtypeStruct((MP, D), jnp.float32),
        scratch_shapes=[pltpu.VMEM((MP, D), jnp.float32)],
    )(p0, p1, a0_col, map_col, emb128, W_u, b_u)


def _cg_body(si_ref, eps_ref, ci_ref, cj_ref, cgp_ref,
             wmu1_ref, bmu1_ref, wmu2_ref, bmu2_ref,
             wsg1_ref, bsg1_ref, wsg2_ref, bsg2_ref,
             wp1a_ref, wp1b_ref, wp1c_ref, bp1_ref, wp2v_ref, bp2v_ref,
             smu_ref, ssig_ref, cgv0_ref, cgv1_ref, cgv2_ref):
    S_I = si_ref[...]
    mu1 = jnp.maximum(jnp.dot(S_I, wmu1_ref[...],
                              preferred_element_type=jnp.float32)
                      + bmu1_ref[...], 0.0)
    S_mu = jnp.dot(mu1, wmu2_ref[...],
                   preferred_element_type=jnp.float32) + bmu2_ref[...]
    sg1 = jnp.maximum(jnp.dot(S_I, wsg1_ref[...],
                              preferred_element_type=jnp.float32)
                      + bsg1_ref[...], 0.0)
    S_logvar = jnp.dot(sg1, wsg2_ref[...],
                       preferred_element_type=jnp.float32) + bsg2_ref[...]
    S_sigma = jnp.exp(S_logvar * 0.5)
    S_lat = eps_ref[...] * S_sigma + S_mu

    ci = ci_ref[...]
    cj = cj_ref[...]
    lane_m = lax.broadcasted_iota(jnp.int32, (E_CG, MP), 1)
    oh_i = (lane_m == ci).astype(jnp.float32)
    oh_j = (lane_m == cj).astype(jnp.float32)
    cgp = cgp_ref[...]
    pi = jnp.dot(oh_i, cgp, preferred_element_type=jnp.float32)
    pj = jnp.dot(oh_j, cgp, preferred_element_type=jnp.float32)
    dvec = pj - pi
    d2 = jnp.sum(dvec * dvec, axis=1, keepdims=True)
    cdist = jnp.sqrt(d2) + 1e-8
    unit = dvec / cdist
    rbf = jnp.exp(-2.0 * (cdist - _centers_row(E_CG)) ** 2)

    Si = jnp.dot(oh_i, S_lat, preferred_element_type=jnp.float32)
    Sj = jnp.dot(oh_j, S_lat, preferred_element_type=jnp.float32)
    pre = (jnp.dot(Si, wp1a_ref[...], preferred_element_type=jnp.float32)
           + jnp.dot(Sj, wp1b_ref[...], preferred_element_type=jnp.float32)
           + jnp.dot(rbf, wp1c_ref[...], preferred_element_type=jnp.float32)
           + bp1_ref[...])
    phi1 = jnp.maximum(pre, 0.0)
    v_w = jnp.dot(phi1, wp2v_ref[...],
                  preferred_element_type=jnp.float32) + bp2v_ref[...]

    lane_d = lax.broadcasted_iota(jnp.int32, (D, 1), 0)
    for c, out in ((0, cgv0_ref), (1, cgv1_ref), (2, cgv2_ref)):
        ec = (lane_d == c).astype(jnp.float32)
        uc = jnp.dot(unit, ec, preferred_element_type=jnp.float32)
        wv = v_w * uc
        out[...] = lax.dot_general(oh_i, wv, (((0,), (0,)), ((), ())),
                                   preferred_element_type=jnp.float32)

    smu_ref[...] = S_mu
    ssig_ref[...] = S_sigma


def _cg_call(S_I, eps_pad, ci_col, cj_col, cgp, weights):
    (W_mu1, b_mu1, W_mu2, b_mu2, W_sg1, b_sg1, W_sg2, b_sg2,
     W_p1a, W_p1b, W_p1c, b_p1, W_p2v, b_p2v) = weights
    out_shape = [jax.ShapeDtypeStruct((MP, D), jnp.float32)] * 5
    return pl.pallas_call(
        _cg_body,
        out_shape=out_shape,
    )(S_I, eps_pad, ci_col, cj_col, cgp,
      W_mu1, b_mu1, W_mu2, b_mu2, W_sg1, b_sg1, W_sg2, b_sg2,
      W_p1a, W_p1b, W_p1c, b_p1, W_p2v, b_p2v)


def _recon_body(cgv_ref, map_ref, cgp_ref, out_ref):
    cgm = map_ref[...]
    lane_m = lax.broadcasted_iota(jnp.int32, (TN, MP), 1)
    onehot = (lane_m == cgm).astype(jnp.float32)
    anchor = jnp.dot(onehot, cgp_ref[...],
                     preferred_element_type=jnp.float32)
    out_ref[...] = cgv_ref[...] + anchor


def _recon_call(cgv_flat_pad, map_col, cgp):
    grid = N // TN
    return pl.pallas_call(
        _recon_body,
        grid=(grid,),
        in_specs=[
            pl.BlockSpec((TN, D), lambda i: (i, 0)),
            pl.BlockSpec((TN, 1), lambda i: (i, 0)),
            pl.BlockSpec((MP, D), lambda i: (0, 0)),
        ],
        out_specs=pl.BlockSpec((TN, D), lambda i: (i, 0)),
        out_shape=jax.ShapeDtypeStruct((N, D), jnp.float32),
    )(cgv_flat_pad, map_col, cgp)


def kernel(nxyz, CG_nxyz, CG_mapping, nbr_list, CG_nbr_list, num_CGs, eps,
           emb, W_f, b_f, W_u, b_u, W_p1, b_p1, W_p2, b_p2,
           W_mu1, b_mu1, W_mu2, b_mu2, W_sg1, b_sg1, W_sg2, b_sg2):
    xyz = nxyz[:, 1:]
    a0_col = nxyz[:, 0:1]
    src = jnp.concatenate(
        [nbr_list[:, 0].astype(jnp.int32), jnp.zeros((E2 - E,), jnp.int32)])
    dst = jnp.concatenate(
        [nbr_list[:, 1].astype(jnp.int32), jnp.full((E2 - E,), N, jnp.int32)])
    src2d = src.reshape(GROUPS, 128)
    dst2d = dst.reshape(GROUPS, 128)

    nxyz16 = jnp.zeros((N, GW), jnp.float32).at[:, :4].set(nxyz)
    emb128 = jnp.zeros((D, D), jnp.float32).at[:VOCAB].set(emb)
    gs, gd = _geom_call(nxyz16, src2d, dst2d)

    msg = _msg_call(gs, gd, emb128, W_f, b_f[None, :])

    zeros_tile = jnp.zeros((NPT, DH), jnp.float32)
    agg0, agg1 = _scatter_call(msg, dst2d, zeros_tile)

    map_col = CG_mapping[:, None].astype(jnp.int32)
    S_I = _atom_call(agg0, agg1, a0_col, map_col, emb128, W_u, b_u[None, :])

    eps_pad = jnp.zeros((MP, D), jnp.float32).at[:M].set(eps)
    cgp = jnp.zeros((MP, D), jnp.float32).at[:M, :3].set(CG_nxyz[:, 1:])
    ci_col = CG_nbr_list[:, 0:1].astype(jnp.int32)
    cj_col = CG_nbr_list[:, 1:2].astype(jnp.int32)
    weights = (W_mu1, b_mu1[None, :], W_mu2, b_mu2[None, :],
               W_sg1, b_sg1[None, :], W_sg2, b_sg2[None, :],
               W_p1[:D], W_p1[D:2 * D], W_p1[2 * D:], b_p1[None, :],
               jnp.zeros((D, D), jnp.float32).at[:, :F_VEC].set(W_p2[:, D:]),
               jnp.zeros((1, D), jnp.float32).at[0, :F_VEC].set(b_p2[D:]))
    S_mu_p, S_sig_p, cgv0, cgv1, cgv2 = _cg_call(
        S_I, eps_pad, ci_col, cj_col, cgp, weights)

    cgv_flat = jnp.stack(
        [cgv0[:M, :F_VEC].reshape(-1),
         cgv1[:M, :F_VEC].reshape(-1),
         cgv2[:M, :F_VEC].reshape(-1)], axis=-1)
    cgv_flat_pad = jnp.zeros((N, D), jnp.float32).at[:, :3].set(cgv_flat)
    recon_pad = _recon_call(cgv_flat_pad, map_col, cgp)

    return (S_mu_p[:M], S_sig_p[:M], xyz, recon_pad[:, :3])


# speedup vs baseline: 1.1141x; 1.0249x over previous
import functools
import jax
import jax.numpy as jnp
from jax import lax
from jax.experimental import pallas as pl
from jax.experimental.pallas import tpu as pltpu
from jax.experimental.pallas import tpu_sc as plsc

N = 10000
M = 200
E = 320000
E_CG = 3200
D = 128
N_RBF = 16
F_VEC = 50
VOCAB = 100
MP = 256
TE = 10240
TN = 1000

_INV_STEP = 15.0 / 5.0

_SC_CORES = 2
_SC_SUBCORES = 16
_SC_WORKERS = _SC_CORES * _SC_SUBCORES
E2 = 327680
GROUPS = E2 // 128
NROWS = N + 16
NPT = NROWS // _SC_SUBCORES


GW = 16
GWO = GW
GCH_G = 4
GCHUNK = GCH_G * 128
EH = E2
GROUPS_H = GROUPS
GPW = GROUPS_H // _SC_WORKERS


def _geom_body(nxyz16_hbm, src2d_hbm, dst2d_hbm, gs_hbm, gd_hbm,
               sidx, didx, rows_s, rows_d, sem_s, sem_d):
    c = lax.axis_index("c")
    s = lax.axis_index("s")
    wgrp = (c * _SC_SUBCORES + s) * GPW

    def body(i, carry):
        gr = wgrp + i * GCH_G
        pltpu.sync_copy(src2d_hbm.at[pl.ds(gr, GCH_G)], sidx)
        pltpu.sync_copy(dst2d_hbm.at[pl.ds(gr, GCH_G)], didx)
        cps = []
        for j in range(GCH_G):
            sl = pl.ds(j * 128, 128)
            cps.append(pltpu.async_copy(
                nxyz16_hbm.at[sidx.at[j]], rows_s.at[sl], sem_s))
            cps.append(pltpu.async_copy(
                nxyz16_hbm.at[didx.at[j]], rows_d.at[sl], sem_d))
        for cp in cps:
            cp.wait()
        pltpu.sync_copy(rows_s, gs_hbm.at[pl.ds(gr * 128, GCHUNK)])
        pltpu.sync_copy(rows_d, gd_hbm.at[pl.ds(gr * 128, GCHUNK)])
        return carry

    lax.fori_loop(0, GPW // GCH_G, body, 0)


def _geom_call(nxyz16, src2d, dst2d):
    mesh = plsc.VectorSubcoreMesh(core_axis_name="c", subcore_axis_name="s")
    f = functools.partial(
        pl.kernel, _geom_body, mesh=mesh,
        compiler_params=pltpu.CompilerParams(use_tc_tiling_on_sc=False),
        out_type=(jax.ShapeDtypeStruct((EH, GWO), jnp.float32),
                  jax.ShapeDtypeStruct((EH, GWO), jnp.float32)),
        scratch_types=[
            pltpu.VMEM((GCH_G, 128), jnp.int32),
            pltpu.VMEM((GCH_G, 128), jnp.int32),
            pltpu.VMEM((GCHUNK, GW), jnp.float32),
            pltpu.VMEM((GCHUNK, GW), jnp.float32),
            pltpu.SemaphoreType.DMA,
            pltpu.SemaphoreType.DMA,
        ],
    )()
    return f(nxyz16, src2d, dst2d)


DH = D // 2
SCH_G = 2
GPT = GROUPS_H // _SC_SUBCORES
NCH = GPT // SCH_G


def _scatter_body(msg_hbm, d2d_hbm, zeros_hbm,
                  agg_hbm, agg, buf0, buf1, idx0, idx1, semf0, semf1):
    c = lax.axis_index("c")
    s = lax.axis_index("s")
    rbase = s * NPT
    col = c * DH
    bufs, idxs, semfs = (buf0, buf1), (idx0, idx1), (semf0, semf1)

    pltpu.sync_copy(zeros_hbm, agg.at[pl.ds(rbase, NPT)])
    plsc.subcore_barrier()

    def fetch(k, b):
        r = s * GPT + k * SCH_G
        return (pltpu.make_async_copy(
                    d2d_hbm.at[pl.ds(r, SCH_G)], idxs[b], semfs[b]),
                pltpu.make_async_copy(
                    msg_hbm.at[pl.ds(r * 128, SCH_G * 128),
                               pl.ds(col, DH)],
                    bufs[b], semfs[b]))

    for b in range(2):
        for cp in fetch(b, b):
            cp.start()

    def body(i, carry):
        for b in range(2):
            k = 2 * i + b
            for cp in fetch(k, b):
                cp.wait()
            for j in range(SCH_G):
                pltpu.sync_copy(bufs[b].at[pl.ds(j * 128, 128)],
                                agg.at[idxs[b].at[j]], add=True)
            kn = jnp.minimum(k + 2, NCH - 1)
            for cp in fetch(kn, b):
                cp.start()
        return carry

    lax.fori_loop(0, NCH // 2, body, 0)
    for b in range(2):
        for cp in fetch(0, b):
            cp.wait()
    plsc.subcore_barrier()
    pltpu.sync_copy(agg.at[pl.ds(rbase, NPT)],
                    agg_hbm.at[c, pl.ds(rbase, NPT)])


def _scatter_call(msg, d2d, zeros_tile):
    mesh = plsc.VectorSubcoreMesh(core_axis_name="c", subcore_axis_name="s")
    f = functools.partial(
        pl.kernel, _scatter_body, mesh=mesh,
        compiler_params=pltpu.CompilerParams(use_tc_tiling_on_sc=False),
        out_type=jax.ShapeDtypeStruct((_SC_CORES, NROWS, DH), jnp.float32),
        scratch_types=[
            pltpu.VMEM_SHARED((NROWS, DH), jnp.float32),
            pltpu.VMEM((SCH_G * 128, DH), jnp.float32),
            pltpu.VMEM((SCH_G * 128, DH), jnp.float32),
            pltpu.VMEM((SCH_G, 128), jnp.int32),
            pltpu.VMEM((SCH_G, 128), jnp.int32),
            pltpu.SemaphoreType.DMA,
            pltpu.SemaphoreType.DMA,
        ],
    )()
    out = f(msg, d2d, zeros_tile)
    return out[0, :N], out[1, :N]


def _centers_row(rows):
    k = lax.broadcasted_iota(jnp.int32, (rows, N_RBF), 1)
    return k.astype(jnp.float32) / _INV_STEP


def _msg_body(gs_ref, gd_ref, emb_ref, wf_ref, bf_ref, out_ref):
    gs = gs_ref[...]
    gd = gd_ref[...]
    dvec = gd[:, 1:4] - gs[:, 1:4]
    d = jnp.sqrt(jnp.sum(dvec * dvec, axis=1, keepdims=True))
    a0 = gs[:, 0:1]
    z = jnp.clip(jnp.abs(a0 * 10.0).astype(jnp.int32), 0, VOCAB - 1)
    lane = lax.broadcasted_iota(jnp.int32, (TE, D), 1)
    onehot = (lane == z).astype(jnp.float32)
    base = jnp.dot(onehot, emb_ref[...], preferred_element_type=jnp.float32)
    rbf = jnp.exp(-2.0 * (d - _centers_row(TE)) ** 2)
    filt = jnp.dot(rbf, wf_ref[...], preferred_element_type=jnp.float32)
    filt = filt + bf_ref[...]
    out_ref[...] = base * filt


def _msg_call(gs, gd, emb128, W_f, b_f):
    grid = EH // TE
    return pl.pallas_call(
        _msg_body,
        grid=(grid,),
        in_specs=[
            pl.BlockSpec((TE, GWO), lambda i: (i, 0)),
            pl.BlockSpec((TE, GWO), lambda i: (i, 0)),
            pl.BlockSpec((D, D), lambda i: (0, 0)),
            pl.BlockSpec((N_RBF, D), lambda i: (0, 0)),
            pl.BlockSpec((1, D), lambda i: (0, 0)),
        ],
        out_specs=pl.BlockSpec((TE, D), lambda i: (i, 0)),
        out_shape=jax.ShapeDtypeStruct((EH, D), jnp.float32),
    )(gs, gd, emb128, W_f, b_f)


def _atom_body(p0_ref, p1_ref, a0_ref, map_ref, emb_ref, wu_ref, bu_ref,
               si_ref, acc_ref):
    step = pl.program_id(0)
    agg = jnp.concatenate([p0_ref[...], p1_ref[...]], axis=1)
    a0 = a0_ref[...]
    z = jnp.clip(jnp.abs(a0 * 10.0).astype(jnp.int32), 0, VOCAB - 1)
    lane = lax.broadcasted_iota(jnp.int32, (TN, D), 1)
    onehot = (lane == z).astype(jnp.float32)
    h0 = jnp.dot(onehot, emb_ref[...], preferred_element_type=jnp.float32)
    u = jnp.dot(agg, wu_ref[...], preferred_element_type=jnp.float32)
    h = h0 + jnp.maximum(u + bu_ref[...], 0.0)
    cgm = map_ref[...]
    lane_m = lax.broadcasted_iota(jnp.int32, (TN, MP), 1)
    onehot_cg = (lane_m == cgm).astype(jnp.float32)
    part = lax.dot_general(onehot_cg, h, (((0,), (0,)), ((), ())),
                           preferred_element_type=jnp.float32)

    @pl.when(step == 0)
    def _():
        acc_ref[...] = jnp.zeros_like(acc_ref)

    acc_ref[...] += part
    si_ref[...] = acc_ref[...]


def _atom_call(p0, p1, a0_col, map_col, emb128, W_u, b_u):
    grid = N // TN
    return pl.pallas_call(
        _atom_body,
        grid=(grid,),
        in_specs=[
            pl.BlockSpec((TN, DH), lambda i: (i, 0)),
            pl.BlockSpec((TN, DH), lambda i: (i, 0)),
            pl.BlockSpec((TN, 1), lambda i: (i, 0)),
            pl.BlockSpec((TN, 1), lambda i: (i, 0)),
            pl.BlockSpec((D, D), lambda i: (0, 0)),
            pl.BlockSpec((D, D), lambda i: (0, 0)),
            pl.BlockSpec((1, D), lambda i: (0, 0)),
        ],
        out_specs=pl.BlockSpec((MP, D), lambda i: (0, 0)),
        out_shape=jax.ShapeDtypeStruct((MP, D), jnp.float32),
        scratch_shapes=[pltpu.VMEM((MP, D), jnp.float32)],
    )(p0, p1, a0_col, map_col, emb128, W_u, b_u)


def _cg_body(si_ref, eps_ref, ci_ref, cj_ref, cgp_ref,
             wmu1_ref, bmu1_ref, wmu2_ref, bmu2_ref,
             wsg1_ref, bsg1_ref, wsg2_ref, bsg2_ref,
             wp1a_ref, wp1b_ref, wp1c_ref, bp1_ref, wp2v_ref, bp2v_ref,
             smu_ref, ssig_ref, cgv0_ref, cgv1_ref, cgv2_ref):
    S_I = si_ref[...]
    mu1 = jnp.maximum(jnp.dot(S_I, wmu1_ref[...],
                              preferred_element_type=jnp.float32)
                      + bmu1_ref[...], 0.0)
    S_mu = jnp.dot(mu1, wmu2_ref[...],
                   preferred_element_type=jnp.float32) + bmu2_ref[...]
    sg1 = jnp.maximum(jnp.dot(S_I, wsg1_ref[...],
                              preferred_element_type=jnp.float32)
                      + bsg1_ref[...], 0.0)
    S_logvar = jnp.dot(sg1, wsg2_ref[...],
                       preferred_element_type=jnp.float32) + bsg2_ref[...]
    S_sigma = jnp.exp(S_logvar * 0.5)
    S_lat = eps_ref[...] * S_sigma + S_mu

    ci = ci_ref[...]
    cj = cj_ref[...]
    lane_m = lax.broadcasted_iota(jnp.int32, (E_CG, MP), 1)
    oh_i = (lane_m == ci).astype(jnp.float32)
    oh_j = (lane_m == cj).astype(jnp.float32)
    cgp = cgp_ref[...]
    pi = jnp.dot(oh_i, cgp, preferred_element_type=jnp.float32)
    pj = jnp.dot(oh_j, cgp, preferred_element_type=jnp.float32)
    dvec = pj - pi
    d2 = jnp.sum(dvec * dvec, axis=1, keepdims=True)
    cdist = jnp.sqrt(d2) + 1e-8
    unit = dvec / cdist
    rbf = jnp.exp(-2.0 * (cdist - _centers_row(E_CG)) ** 2)

    Si = jnp.dot(oh_i, S_lat, preferred_element_type=jnp.float32)
    Sj = jnp.dot(oh_j, S_lat, preferred_element_type=jnp.float32)
    pre = (jnp.dot(Si, wp1a_ref[...], preferred_element_type=jnp.float32)
           + jnp.dot(Sj, wp1b_ref[...], preferred_element_type=jnp.float32)
           + jnp.dot(rbf, wp1c_ref[...], preferred_element_type=jnp.float32)
           + bp1_ref[...])
    phi1 = jnp.maximum(pre, 0.0)
    v_w = jnp.dot(phi1, wp2v_ref[...],
                  preferred_element_type=jnp.float32) + bp2v_ref[...]

    lane_d = lax.broadcasted_iota(jnp.int32, (D, 1), 0)
    for c, out in ((0, cgv0_ref), (1, cgv1_ref), (2, cgv2_ref)):
        ec = (lane_d == c).astype(jnp.float32)
        uc = jnp.dot(unit, ec, preferred_element_type=jnp.float32)
        wv = v_w * uc
        out[...] = lax.dot_general(oh_i, wv, (((0,), (0,)), ((), ())),
                                   preferred_element_type=jnp.float32)

    smu_ref[...] = S_mu
    ssig_ref[...] = S_sigma


def _cg_call(S_I, eps_pad, ci_col, cj_col, cgp, weights):
    (W_mu1, b_mu1, W_mu2, b_mu2, W_sg1, b_sg1, W_sg2, b_sg2,
     W_p1a, W_p1b, W_p1c, b_p1, W_p2v, b_p2v) = weights
    out_shape = [jax.ShapeDtypeStruct((MP, D), jnp.float32)] * 5
    return pl.pallas_call(
        _cg_body,
        out_shape=out_shape,
    )(S_I, eps_pad, ci_col, cj_col, cgp,
      W_mu1, b_mu1, W_mu2, b_mu2, W_sg1, b_sg1, W_sg2, b_sg2,
      W_p1a, W_p1b, W_p1c, b_p1, W_p2v, b_p2v)


def _recon_body(cgv_ref, map_ref, cgp_ref, out_ref):
    cgm = map_ref[...]
    lane_m = lax.broadcasted_iota(jnp.int32, (TN, MP), 1)
    onehot = (lane_m == cgm).astype(jnp.float32)
    anchor = jnp.dot(onehot, cgp_ref[...],
                     preferred_element_type=jnp.float32)
    out_ref[...] = cgv_ref[...] + anchor


def _recon_call(cgv_flat_pad, map_col, cgp):
    grid = N // TN
    return pl.pallas_call(
        _recon_body,
        grid=(grid,),
        in_specs=[
            pl.BlockSpec((TN, D), lambda i: (i, 0)),
            pl.BlockSpec((TN, 1), lambda i: (i, 0)),
            pl.BlockSpec((MP, D), lambda i: (0, 0)),
        ],
        out_specs=pl.BlockSpec((TN, D), lambda i: (i, 0)),
        out_shape=jax.ShapeDtypeStruct((N, D), jnp.float32),
    )(cgv_flat_pad, map_col, cgp)


def kernel(nxyz, CG_nxyz, CG_mapping, nbr_list, CG_nbr_list, num_CGs, eps,
           emb, W_f, b_f, W_u, b_u, W_p1, b_p1, W_p2, b_p2,
           W_mu1, b_mu1, W_mu2, b_mu2, W_sg1, b_sg1, W_sg2, b_sg2):
    xyz = nxyz[:, 1:]
    a0_col = nxyz[:, 0:1]
    src = jnp.concatenate(
        [nbr_list[:, 0].astype(jnp.int32), jnp.zeros((E2 - E,), jnp.int32)])
    dst = jnp.concatenate(
        [nbr_list[:, 1].astype(jnp.int32), jnp.full((E2 - E,), N, jnp.int32)])
    src2d = src.reshape(GROUPS, 128)
    dst2d = dst.reshape(GROUPS, 128)

    nxyz16 = jnp.zeros((N, GW), jnp.float32).at[:, :4].set(nxyz)
    emb128 = jnp.zeros((D, D), jnp.float32).at[:VOCAB].set(emb)
    gs, gd = _geom_call(nxyz16, src2d, dst2d)

    msg = _msg_call(gs, gd, emb128, W_f, b_f[None, :])

    zeros_tile = jnp.zeros((NPT, DH), jnp.float32)
    agg0, agg1 = _scatter_call(msg, dst2d, zeros_tile)

    map_col = CG_mapping[:, None].astype(jnp.int32)
    S_I = _atom_call(agg0, agg1, a0_col, map_col, emb128, W_u, b_u[None, :])

    eps_pad = jnp.zeros((MP, D), jnp.float32).at[:M].set(eps)
    cgp = jnp.zeros((MP, D), jnp.float32).at[:M, :3].set(CG_nxyz[:, 1:])
    ci_col = CG_nbr_list[:, 0:1].astype(jnp.int32)
    cj_col = CG_nbr_list[:, 1:2].astype(jnp.int32)
    weights = (W_mu1, b_mu1[None, :], W_mu2, b_mu2[None, :],
               W_sg1, b_sg1[None, :], W_sg2, b_sg2[None, :],
               W_p1[:D], W_p1[D:2 * D], W_p1[2 * D:], b_p1[None, :],
               jnp.zeros((D, D), jnp.float32).at[:, :F_VEC].set(W_p2[:, D:]),
               jnp.zeros((1, D), jnp.float32).at[0, :F_VEC].set(b_p2[D:]))
    S_mu_p, S_sig_p, cgv0, cgv1, cgv2 = _cg_call(
        S_I, eps_pad, ci_col, cj_col, cgp, weights)

    cgv_flat = jnp.stack(
        [cgv0[:M, :F_VEC].reshape(-1),
         cgv1[:M, :F_VEC].reshape(-1),
         cgv2[:M, :F_VEC].reshape(-1)], axis=-1)
    cgv_flat_pad = jnp.zeros((N, D), jnp.float32).at[:, :3].set(cgv_flat)
    recon_pad = _recon_call(cgv_flat_pad, map_col, cgp)

    return (S_mu_p[:M], S_sig_p[:M], xyz, recon_pad[:, :3])


# speedup vs baseline: 1.1325x; 1.0165x over previous
import functools
import jax
import jax.numpy as jnp
from jax import lax
from jax.experimental import pallas as pl
from jax.experimental.pallas import tpu as pltpu
from jax.experimental.pallas import tpu_sc as plsc

N = 10000
M = 200
E = 320000
E_CG = 3200
D = 128
N_RBF = 16
F_VEC = 50
VOCAB = 100
MP = 256
TE = 16384
TN = 2000

_INV_STEP = 15.0 / 5.0

_SC_CORES = 2
_SC_SUBCORES = 16
_SC_WORKERS = _SC_CORES * _SC_SUBCORES
E2 = 327680
GROUPS = E2 // 128
NROWS = N + 16
NPT = NROWS // _SC_SUBCORES


GW = 16
GWO = GW
GCH_G = 4
GCHUNK = GCH_G * 128
EH = E2
GROUPS_H = GROUPS
GPW = GROUPS_H // _SC_WORKERS


def _geom_body(nxyz16_hbm, src2d_hbm, dst2d_hbm, gs_hbm, gd_hbm,
               sidx, didx, rows_s, rows_d, sem_s, sem_d):
    c = lax.axis_index("c")
    s = lax.axis_index("s")
    wgrp = (c * _SC_SUBCORES + s) * GPW

    def body(i, carry):
        gr = wgrp + i * GCH_G
        pltpu.sync_copy(src2d_hbm.at[pl.ds(gr, GCH_G)], sidx)
        pltpu.sync_copy(dst2d_hbm.at[pl.ds(gr, GCH_G)], didx)
        cps = []
        for j in range(GCH_G):
            sl = pl.ds(j * 128, 128)
            cps.append(pltpu.async_copy(
                nxyz16_hbm.at[sidx.at[j]], rows_s.at[sl], sem_s))
            cps.append(pltpu.async_copy(
                nxyz16_hbm.at[didx.at[j]], rows_d.at[sl], sem_d))
        for cp in cps:
            cp.wait()
        pltpu.sync_copy(rows_s, gs_hbm.at[pl.ds(gr * 128, GCHUNK)])
        pltpu.sync_copy(rows_d, gd_hbm.at[pl.ds(gr * 128, GCHUNK)])
        return carry

    lax.fori_loop(0, GPW // GCH_G, body, 0)


def _geom_call(nxyz16, src2d, dst2d):
    mesh = plsc.VectorSubcoreMesh(core_axis_name="c", subcore_axis_name="s")
    f = functools.partial(
        pl.kernel, _geom_body, mesh=mesh,
        compiler_params=pltpu.CompilerParams(use_tc_tiling_on_sc=False),
        out_type=(jax.ShapeDtypeStruct((EH, GWO), jnp.float32),
                  jax.ShapeDtypeStruct((EH, GWO), jnp.float32)),
        scratch_types=[
            pltpu.VMEM((GCH_G, 128), jnp.int32),
            pltpu.VMEM((GCH_G, 128), jnp.int32),
            pltpu.VMEM((GCHUNK, GW), jnp.float32),
            pltpu.VMEM((GCHUNK, GW), jnp.float32),
            pltpu.SemaphoreType.DMA,
            pltpu.SemaphoreType.DMA,
        ],
    )()
    return f(nxyz16, src2d, dst2d)


DH = D // 2
SCH_G = 2
GPT = GROUPS_H // _SC_SUBCORES
NCH = GPT // SCH_G


def _scatter_body(msg_hbm, d2d_hbm, zeros_hbm,
                  agg_hbm, agg, buf0, buf1, idx0, idx1, semf0, semf1):
    c = lax.axis_index("c")
    s = lax.axis_index("s")
    rbase = s * NPT
    col = c * DH
    bufs, idxs, semfs = (buf0, buf1), (idx0, idx1), (semf0, semf1)

    pltpu.sync_copy(zeros_hbm, agg.at[pl.ds(rbase, NPT)])
    plsc.subcore_barrier()

    def fetch(k, b):
        r = s * GPT + k * SCH_G
        return (pltpu.make_async_copy(
                    d2d_hbm.at[pl.ds(r, SCH_G)], idxs[b], semfs[b]),
                pltpu.make_async_copy(
                    msg_hbm.at[pl.ds(r * 128, SCH_G * 128),
                               pl.ds(col, DH)],
                    bufs[b], semfs[b]))

    for b in range(2):
        for cp in fetch(b, b):
            cp.start()

    def body(i, carry):
        for b in range(2):
            k = 2 * i + b
            for cp in fetch(k, b):
                cp.wait()
            for j in range(SCH_G):
                pltpu.sync_copy(bufs[b].at[pl.ds(j * 128, 128)],
                                agg.at[idxs[b].at[j]], add=True)
            kn = jnp.minimum(k + 2, NCH - 1)
            for cp in fetch(kn, b):
                cp.start()
        return carry

    lax.fori_loop(0, NCH // 2, body, 0)
    for b in range(2):
        for cp in fetch(0, b):
            cp.wait()
    plsc.subcore_barrier()
    pltpu.sync_copy(agg.at[pl.ds(rbase, NPT)],
                    agg_hbm.at[c, pl.ds(rbase, NPT)])


def _scatter_call(msg, d2d, zeros_tile):
    mesh = plsc.VectorSubcoreMesh(core_axis_name="c", subcore_axis_name="s")
    f = functools.partial(
        pl.kernel, _scatter_body, mesh=mesh,
        compiler_params=pltpu.CompilerParams(use_tc_tiling_on_sc=False),
        out_type=jax.ShapeDtypeStruct((_SC_CORES, NROWS, DH), jnp.float32),
        scratch_types=[
            pltpu.VMEM_SHARED((NROWS, DH), jnp.float32),
            pltpu.VMEM((SCH_G * 128, DH), jnp.float32),
            pltpu.VMEM((SCH_G * 128, DH), jnp.float32),
            pltpu.VMEM((SCH_G, 128), jnp.int32),
            pltpu.VMEM((SCH_G, 128), jnp.int32),
            pltpu.SemaphoreType.DMA,
            pltpu.SemaphoreType.DMA,
        ],
    )()
    out = f(msg, d2d, zeros_tile)
    return out[0, :N], out[1, :N]


def _centers_row(rows):
    k = lax.broadcasted_iota(jnp.int32, (rows, N_RBF), 1)
    return k.astype(jnp.float32) / _INV_STEP


def _msg_body(gs_ref, gd_ref, emb_ref, wf_ref, bf_ref, out_ref):
    gs = gs_ref[...]
    gd = gd_ref[...]
    dvec = gd[:, 1:4] - gs[:, 1:4]
    d = jnp.sqrt(jnp.sum(dvec * dvec, axis=1, keepdims=True))
    a0 = gs[:, 0:1]
    z = jnp.clip(jnp.abs(a0 * 10.0).astype(jnp.int32), 0, VOCAB - 1)
    lane = lax.broadcasted_iota(jnp.int32, (TE, D), 1)
    onehot = (lane == z).astype(jnp.float32)
    base = jnp.dot(onehot, emb_ref[...], preferred_element_type=jnp.float32)
    rbf = jnp.exp(-2.0 * (d - _centers_row(TE)) ** 2)
    filt = jnp.dot(rbf, wf_ref[...], preferred_element_type=jnp.float32)
    filt = filt + bf_ref[...]
    out_ref[...] = base * filt


def _msg_call(gs, gd, emb128, W_f, b_f):
    grid = EH // TE
    return pl.pallas_call(
        _msg_body,
        grid=(grid,),
        in_specs=[
            pl.BlockSpec((TE, GWO), lambda i: (i, 0)),
            pl.BlockSpec((TE, GWO), lambda i: (i, 0)),
            pl.BlockSpec((D, D), lambda i: (0, 0)),
            pl.BlockSpec((N_RBF, D), lambda i: (0, 0)),
            pl.BlockSpec((1, D), lambda i: (0, 0)),
        ],
        out_specs=pl.BlockSpec((TE, D), lambda i: (i, 0)),
        out_shape=jax.ShapeDtypeStruct((EH, D), jnp.float32),
    )(gs, gd, emb128, W_f, b_f)


def _atom_body(p0_ref, p1_ref, a0_ref, map_ref, emb_ref, wu_ref, bu_ref,
               si_ref, acc_ref):
    step = pl.program_id(0)
    agg = jnp.concatenate([p0_ref[...], p1_ref[...]], axis=1)
    a0 = a0_ref[...]
    z = jnp.clip(jnp.abs(a0 * 10.0).astype(jnp.int32), 0, VOCAB - 1)
    lane = lax.broadcasted_iota(jnp.int32, (TN, D), 1)
    onehot = (lane == z).astype(jnp.float32)
    h0 = jnp.dot(onehot, emb_ref[...], preferred_element_type=jnp.float32)
    u = jnp.dot(agg, wu_ref[...], preferred_element_type=jnp.float32)
    h = h0 + jnp.maximum(u + bu_ref[...], 0.0)
    cgm = map_ref[...]
    lane_m = lax.broadcasted_iota(jnp.int32, (TN, MP), 1)
    onehot_cg = (lane_m == cgm).astype(jnp.float32)
    part = lax.dot_general(onehot_cg, h, (((0,), (0,)), ((), ())),
                           preferred_element_type=jnp.float32)

    @pl.when(step == 0)
    def _():
        acc_ref[...] = jnp.zeros_like(acc_ref)

    acc_ref[...] += part
    si_ref[...] = acc_ref[...]


def _atom_call(p0, p1, a0_col, map_col, emb128, W_u, b_u):
    grid = N // TN
    return pl.pallas_call(
        _atom_body,
        grid=(grid,),
        in_specs=[
            pl.BlockSpec((TN, DH), lambda i: (i, 0)),
            pl.BlockSpec((TN, DH), lambda i: (i, 0)),
            pl.BlockSpec((TN, 1), lambda i: (i, 0)),
            pl.BlockSpec((TN, 1), lambda i: (i, 0)),
            pl.BlockSpec((D, D), lambda i: (0, 0)),
            pl.BlockSpec((D, D), lambda i: (0, 0)),
            pl.BlockSpec((1, D), lambda i: (0, 0)),
        ],
        out_specs=pl.BlockSpec((MP, D), lambda i: (0, 0)),
        out_shape=jax.ShapeDtypeStruct((MP, D), jnp.float32),
        scratch_shapes=[pltpu.VMEM((MP, D), jnp.float32)],
    )(p0, p1, a0_col, map_col, emb128, W_u, b_u)


def _cg_body(si_ref, eps_ref, ci_ref, cj_ref, cgp_ref,
             wmu1_ref, bmu1_ref, wmu2_ref, bmu2_ref,
             wsg1_ref, bsg1_ref, wsg2_ref, bsg2_ref,
             wp1a_ref, wp1b_ref, wp1c_ref, bp1_ref, wp2v_ref, bp2v_ref,
             smu_ref, ssig_ref, cgv0_ref, cgv1_ref, cgv2_ref):
    S_I = si_ref[...]
    mu1 = jnp.maximum(jnp.dot(S_I, wmu1_ref[...],
                              preferred_element_type=jnp.float32)
                      + bmu1_ref[...], 0.0)
    S_mu = jnp.dot(mu1, wmu2_ref[...],
                   preferred_element_type=jnp.float32) + bmu2_ref[...]
    sg1 = jnp.maximum(jnp.dot(S_I, wsg1_ref[...],
                              preferred_element_type=jnp.float32)
                      + bsg1_ref[...], 0.0)
    S_logvar = jnp.dot(sg1, wsg2_ref[...],
                       preferred_element_type=jnp.float32) + bsg2_ref[...]
    S_sigma = jnp.exp(S_logvar * 0.5)
    S_lat = eps_ref[...] * S_sigma + S_mu

    ci = ci_ref[...]
    cj = cj_ref[...]
    lane_m = lax.broadcasted_iota(jnp.int32, (E_CG, MP), 1)
    oh_i = (lane_m == ci).astype(jnp.float32)
    oh_j = (lane_m == cj).astype(jnp.float32)
    cgp = cgp_ref[...]
    pi = jnp.dot(oh_i, cgp, preferred_element_type=jnp.float32)
    pj = jnp.dot(oh_j, cgp, preferred_element_type=jnp.float32)
    dvec = pj - pi
    d2 = jnp.sum(dvec * dvec, axis=1, keepdims=True)
    cdist = jnp.sqrt(d2) + 1e-8
    unit = dvec / cdist
    rbf = jnp.exp(-2.0 * (cdist - _centers_row(E_CG)) ** 2)

    Si = jnp.dot(oh_i, S_lat, preferred_element_type=jnp.float32)
    Sj = jnp.dot(oh_j, S_lat, preferred_element_type=jnp.float32)
    pre = (jnp.dot(Si, wp1a_ref[...], preferred_element_type=jnp.float32)
           + jnp.dot(Sj, wp1b_ref[...], preferred_element_type=jnp.float32)
           + jnp.dot(rbf, wp1c_ref[...], preferred_element_type=jnp.float32)
           + bp1_ref[...])
    phi1 = jnp.maximum(pre, 0.0)
    v_w = jnp.dot(phi1, wp2v_ref[...],
                  preferred_element_type=jnp.float32) + bp2v_ref[...]

    lane_d = lax.broadcasted_iota(jnp.int32, (D, 1), 0)
    for c, out in ((0, cgv0_ref), (1, cgv1_ref), (2, cgv2_ref)):
        ec = (lane_d == c).astype(jnp.float32)
        uc = jnp.dot(unit, ec, preferred_element_type=jnp.float32)
        wv = v_w * uc
        out[...] = lax.dot_general(oh_i, wv, (((0,), (0,)), ((), ())),
                                   preferred_element_type=jnp.float32)

    smu_ref[...] = S_mu
    ssig_ref[...] = S_sigma


def _cg_call(S_I, eps_pad, ci_col, cj_col, cgp, weights):
    (W_mu1, b_mu1, W_mu2, b_mu2, W_sg1, b_sg1, W_sg2, b_sg2,
     W_p1a, W_p1b, W_p1c, b_p1, W_p2v, b_p2v) = weights
    out_shape = [jax.ShapeDtypeStruct((MP, D), jnp.float32)] * 5
    return pl.pallas_call(
        _cg_body,
        out_shape=out_shape,
    )(S_I, eps_pad, ci_col, cj_col, cgp,
      W_mu1, b_mu1, W_mu2, b_mu2, W_sg1, b_sg1, W_sg2, b_sg2,
      W_p1a, W_p1b, W_p1c, b_p1, W_p2v, b_p2v)


def _recon_body(cgv_ref, map_ref, cgp_ref, out_ref):
    cgm = map_ref[...]
    lane_m = lax.broadcasted_iota(jnp.int32, (TN, MP), 1)
    onehot = (lane_m == cgm).astype(jnp.float32)
    anchor = jnp.dot(onehot, cgp_ref[...],
                     preferred_element_type=jnp.float32)
    out_ref[...] = cgv_ref[...] + anchor


def _recon_call(cgv_flat_pad, map_col, cgp):
    grid = N // TN
    return pl.pallas_call(
        _recon_body,
        grid=(grid,),
        in_specs=[
            pl.BlockSpec((TN, D), lambda i: (i, 0)),
            pl.BlockSpec((TN, 1), lambda i: (i, 0)),
            pl.BlockSpec((MP, D), lambda i: (0, 0)),
        ],
        out_specs=pl.BlockSpec((TN, D), lambda i: (i, 0)),
        out_shape=jax.ShapeDtypeStruct((N, D), jnp.float32),
    )(cgv_flat_pad, map_col, cgp)


def kernel(nxyz, CG_nxyz, CG_mapping, nbr_list, CG_nbr_list, num_CGs, eps,
           emb, W_f, b_f, W_u, b_u, W_p1, b_p1, W_p2, b_p2,
           W_mu1, b_mu1, W_mu2, b_mu2, W_sg1, b_sg1, W_sg2, b_sg2):
    xyz = nxyz[:, 1:]
    a0_col = nxyz[:, 0:1]
    src = jnp.concatenate(
        [nbr_list[:, 0].astype(jnp.int32), jnp.zeros((E2 - E,), jnp.int32)])
    dst = jnp.concatenate(
        [nbr_list[:, 1].astype(jnp.int32), jnp.full((E2 - E,), N, jnp.int32)])
    src2d = src.reshape(GROUPS, 128)
    dst2d = dst.reshape(GROUPS, 128)

    nxyz16 = jnp.zeros((N, GW), jnp.float32).at[:, :4].set(nxyz)
    emb128 = jnp.zeros((D, D), jnp.float32).at[:VOCAB].set(emb)
    gs, gd = _geom_call(nxyz16, src2d, dst2d)

    msg = _msg_call(gs, gd, emb128, W_f, b_f[None, :])

    zeros_tile = jnp.zeros((NPT, DH), jnp.float32)
    agg0, agg1 = _scatter_call(msg, dst2d, zeros_tile)

    map_col = CG_mapping[:, None].astype(jnp.int32)
    S_I = _atom_call(agg0, agg1, a0_col, map_col, emb128, W_u, b_u[None, :])

    eps_pad = jnp.zeros((MP, D), jnp.float32).at[:M].set(eps)
    cgp = jnp.zeros((MP, D), jnp.float32).at[:M, :3].set(CG_nxyz[:, 1:])
    ci_col = CG_nbr_list[:, 0:1].astype(jnp.int32)
    cj_col = CG_nbr_list[:, 1:2].astype(jnp.int32)
    weights = (W_mu1, b_mu1[None, :], W_mu2, b_mu2[None, :],
               W_sg1, b_sg1[None, :], W_sg2, b_sg2[None, :],
               W_p1[:D], W_p1[D:2 * D], W_p1[2 * D:], b_p1[None, :],
               jnp.zeros((D, D), jnp.float32).at[:, :F_VEC].set(W_p2[:, D:]),
               jnp.zeros((1, D), jnp.float32).at[0, :F_VEC].set(b_p2[D:]))
    S_mu_p, S_sig_p, cgv0, cgv1, cgv2 = _cg_call(
        S_I, eps_pad, ci_col, cj_col, cgp, weights)

    cgv_flat = jnp.stack(
        [cgv0[:M, :F_VEC].reshape(-1),
         cgv1[:M, :F_VEC].reshape(-1),
         cgv2[:M, :F_VEC].reshape(-1)], axis=-1)
    cgv_flat_pad = jnp.zeros((N, D), jnp.float32).at[:, :3].set(cgv_flat)
    recon_pad = _recon_call(cgv_flat_pad, map_col, cgp)

    return (S_mu_p[:M], S_sig_p[:M], xyz, recon_pad[:, :3])
